# baseline (speedup 1.0000x reference)
import functools

import jax
import jax.numpy as jnp
from jax import lax
from jax.experimental import pallas as pl
from jax.experimental.pallas import tpu as pltpu
from jax.experimental.pallas import tpu_sc as plsc


def _mse_body(xr_ref, x_ref, out_ref, *, nb):
    b = pl.program_id(0)
    c = pl.program_id(1)

    @pl.when((b == 0) & (c == 0))
    def _():
        out_ref[...] = jnp.zeros_like(out_ref)

    xb = x_ref[0]
    d0 = xr_ref[0, 0] - xb
    d1 = xr_ref[1, 0] - xb
    s0 = jnp.sum(d0 * d0)
    s1 = jnp.sum(d1 * d1)
    lane = lax.broadcasted_iota(jnp.int32, (1, 2 * nb), 1)
    out_ref[...] += jnp.where(lane == b, s0, 0.0) + jnp.where(
        lane == b + nb, s1, 0.0)


def _combine_sc_body(sums_hbm, out_hbm, sums_v, out_v, *, inv):
    cid = lax.axis_index("c")
    sid = lax.axis_index("s")

    @pl.when((cid == 0) & (sid == 0))
    def _():
        pltpu.sync_copy(sums_hbm, sums_v)
        lanes = lax.iota(jnp.int32, 16)
        b_idx = lanes & 7
        grp = lanes >> 3
        v = sums_v[0, :]

        def shuf(a, k):
            return a.at[lanes ^ k].get(mode="promise_in_bounds")

        def allsum(a):
            for k in (1, 2, 4, 8):
                a = a + shuf(a, k)
            return a

        def argmax_last(val, idx):
            for k in (1, 2, 4, 8):
                vo, io = shuf(val, k), shuf(idx, k)
                take = (vo > val) | ((vo == val) & (io > idx))
                val = jnp.where(take, vo, val)
                idx = jnp.where(take, io, idx)
            return idx

        neg = jnp.full((16,), float("-inf"), jnp.float32)
        zf = jnp.zeros((16,), jnp.float32)
        g0 = grp == 0
        g1 = grp == 1
        j0 = argmax_last(jnp.where(g0, v, neg), b_idx)
        j1 = argmax_last(jnp.where(g1, v, neg), b_idx)
        s0 = allsum(jnp.where(g0, v, zf))
        s1 = allsum(jnp.where(g1, v, zf))
        d0 = allsum(jnp.where(g0 & (b_idx == j1), v, zf))
        d1 = allsum(jnp.where(g1 & (b_idx == j0), v, zf))
        loss = (s0 - d0 + s1 - d1) * inv
        out_v[...] = loss
        pltpu.sync_copy(out_v, out_hbm)


def _pick_chunk(c0, h, w, budget_bytes=3400000):
    best = 1
    for g in range(1, c0 + 1):
        if c0 % g == 0 and g * h * w * 4 <= budget_bytes:
            best = g
    return best


def kernel(xr, x):
    B, C0, H, W = x.shape
    N = C0 * H * W
    G = _pick_chunk(C0, H, W)
    C = C0 // G
    rem = int(B * 0.9)
    inv = 1.0 / (rem * N)

    sums = pl.pallas_call(
        functools.partial(_mse_body, nb=B),
        grid=(B, C),
        in_specs=[
            pl.BlockSpec((2, 1, G, H, W), lambda b, c: (0, b, c, 0, 0)),
            pl.BlockSpec((1, G, H, W), lambda b, c: (b, c, 0, 0)),
        ],
        out_specs=pl.BlockSpec((1, 2 * B), lambda b, c: (0, 0)),
        out_shape=jax.ShapeDtypeStruct((1, 2 * B), jnp.float32),
    )(xr, x)

    mesh = plsc.VectorSubcoreMesh(core_axis_name="c", subcore_axis_name="s")
    combine = pl.kernel(
        functools.partial(_combine_sc_body, inv=inv),
        mesh=mesh,
        out_type=jax.ShapeDtypeStruct((16,), jnp.float32),
        scratch_types=[
            pltpu.VMEM((1, 2 * B), jnp.float32),
            pltpu.VMEM((16,), jnp.float32),
        ],
    )
    loss16 = combine(sums)
    return loss16[0]
